# baseline (speedup 1.0000x reference)
import jax
import jax.numpy as jnp
from jax import lax
from jax.experimental import pallas as pl
from jax.experimental.pallas import tpu as pltpu
from jax.experimental.pallas import tpu_sc as plsc

B = 16384
NROWS = 1000000
GMF_DIM = 16
MLP_DIM = 32
NC = 2
NS = 16
NW = NC * NS
BPW = B // NW
CHUNK = 128
NCHUNK = BPW // CHUNK
IDX_ROWS = B // CHUNK


def _sc_gather_body(u2, i2, gue_f, gie_f, gub, gib, mue_f, mie_f,
                    gu_o, gi_o, bu_o, bi_o, mu_o, mi_o,
                    idx_u, idx_i, gu_vt, gi_vt, bu_v, bi_v, mu_vt, mi_vt,
                    sem):
    c = lax.axis_index("c")
    s = lax.axis_index("s")
    wid = s * NC + c
    r0 = wid * NCHUNK
    b0 = wid * BPW

    pltpu.sync_copy(u2.at[pl.ds(r0, NCHUNK)], idx_u)
    pltpu.sync_copy(i2.at[pl.ds(r0, NCHUNK)], idx_i)

    def gmf_feature(f, carry):
        off = f * NROWS
        for j in range(NCHUNK):
            sl = pl.ds(j * CHUNK, CHUNK)
            pltpu.async_copy(gue_f.at[pl.ds(off, NROWS)].at[idx_u.at[j]],
                             gu_vt.at[f, sl], sem)
            pltpu.async_copy(gie_f.at[pl.ds(off, NROWS)].at[idx_i.at[j]],
                             gi_vt.at[f, sl], sem)
        return carry

    lax.fori_loop(0, GMF_DIM, gmf_feature, 0)

    def mlp_feature(f, carry):
        off = f * NROWS
        for j in range(NCHUNK):
            sl = pl.ds(j * CHUNK, CHUNK)
            pltpu.async_copy(mue_f.at[pl.ds(off, NROWS)].at[idx_u.at[j]],
                             mu_vt.at[f, sl], sem)
            pltpu.async_copy(mie_f.at[pl.ds(off, NROWS)].at[idx_i.at[j]],
                             mi_vt.at[f, sl], sem)
        return carry

    lax.fori_loop(0, MLP_DIM, mlp_feature, 0)

    for j in range(NCHUNK):
        sl = pl.ds(j * CHUNK, CHUNK)
        pltpu.async_copy(gub.at[idx_u.at[j]], bu_v.at[sl], sem)
        pltpu.async_copy(gib.at[idx_i.at[j]], bi_v.at[sl], sem)

    bsl = pl.ds(b0, BPW)
    pltpu.make_async_copy(gu_o.at[:, bsl], gu_vt, sem).wait()
    pltpu.make_async_copy(gi_o.at[:, bsl], gi_vt, sem).wait()
    pltpu.make_async_copy(mu_o.at[:, bsl], mu_vt, sem).wait()
    pltpu.make_async_copy(mi_o.at[:, bsl], mi_vt, sem).wait()
    pltpu.make_async_copy(bu_o.at[0, bsl], bu_v, sem).wait()
    pltpu.make_async_copy(bi_o.at[0, bsl], bi_v, sem).wait()

    pltpu.sync_copy(gu_vt, gu_o.at[:, bsl])
    pltpu.sync_copy(gi_vt, gi_o.at[:, bsl])
    pltpu.sync_copy(mu_vt, mu_o.at[:, bsl])
    pltpu.sync_copy(mi_vt, mi_o.at[:, bsl])
    pltpu.sync_copy(bu_v, bu_o.at[0, bsl])
    pltpu.sync_copy(bi_v, bi_o.at[0, bsl])


@jax.jit
def _sc_gather(u2, i2, gue_f, gie_f, gub, gib, mue_f, mie_f):
    mesh = plsc.VectorSubcoreMesh(core_axis_name="c", subcore_axis_name="s")
    f = pl.kernel(
        _sc_gather_body,
        out_type=[
            jax.ShapeDtypeStruct((GMF_DIM, B), jnp.float32),
            jax.ShapeDtypeStruct((GMF_DIM, B), jnp.float32),
            jax.ShapeDtypeStruct((1, B), jnp.float32),
            jax.ShapeDtypeStruct((1, B), jnp.float32),
            jax.ShapeDtypeStruct((MLP_DIM, B), jnp.float32),
            jax.ShapeDtypeStruct((MLP_DIM, B), jnp.float32),
        ],
        mesh=mesh,
        scratch_types=[
            pltpu.VMEM((NCHUNK, CHUNK), jnp.int32),
            pltpu.VMEM((NCHUNK, CHUNK), jnp.int32),
            pltpu.VMEM((GMF_DIM, BPW), jnp.float32),
            pltpu.VMEM((GMF_DIM, BPW), jnp.float32),
            pltpu.VMEM((BPW,), jnp.float32),
            pltpu.VMEM((BPW,), jnp.float32),
            pltpu.VMEM((MLP_DIM, BPW), jnp.float32),
            pltpu.VMEM((MLP_DIM, BPW), jnp.float32),
            pltpu.SemaphoreType.DMA,
        ],
        compiler_params=pltpu.CompilerParams(use_tc_tiling_on_sc=False),
    )
    return f(u2, i2, gue_f, gie_f, gub, gib, mue_f, mie_f)


BLK = 2048


def _tc_body(gu, gi, bu, bi, mu, mi, w1a, w1b, b1, w2, b2, wfh, sc, out):
    h = jnp.dot(w1a[...], mu[...], preferred_element_type=jnp.float32)
    h += jnp.dot(w1b[...], mi[...], preferred_element_type=jnp.float32)
    h = jnp.maximum(h + b1[...], 0.0)
    h = jnp.maximum(jnp.dot(w2[...], h, preferred_element_type=jnp.float32)
                    + b2[...], 0.0)
    gmf = (jnp.sum(gu[...] * gi[...], axis=0, keepdims=True)
           + bu[...] + bi[...])
    scv = sc[...]
    out[...] = (gmf * scv[0:1, 0:1]
                + jnp.dot(wfh[...], h, preferred_element_type=jnp.float32)
                + scv[0:1, 1:2])


@jax.jit
def _tc_mlp(gu, gi, bu, bi, mu, mi, w1a, w1b, b1, w2, b2, wfh, sc):
    grid = B // BLK
    full = lambda i: (0, 0)
    blk_col = lambda i: (0, i)
    return pl.pallas_call(
        _tc_body,
        grid=(grid,),
        in_specs=[
            pl.BlockSpec((GMF_DIM, BLK), blk_col),
            pl.BlockSpec((GMF_DIM, BLK), blk_col),
            pl.BlockSpec((1, BLK), blk_col),
            pl.BlockSpec((1, BLK), blk_col),
            pl.BlockSpec((MLP_DIM, BLK), blk_col),
            pl.BlockSpec((MLP_DIM, BLK), blk_col),
            pl.BlockSpec((32, MLP_DIM), full),
            pl.BlockSpec((32, MLP_DIM), full),
            pl.BlockSpec((32, 1), full),
            pl.BlockSpec((16, 32), full),
            pl.BlockSpec((16, 1), full),
            pl.BlockSpec((1, 16), full),
            pl.BlockSpec((1, 2), full),
        ],
        out_specs=pl.BlockSpec((1, BLK), blk_col),
        out_shape=jax.ShapeDtypeStruct((1, B), jnp.float32),
    )(gu, gi, bu, bi, mu, mi, w1a, w1b, b1, w2, b2, wfh, sc)


def kernel(user_ids, item_ids, gmf_user_emb, gmf_item_emb, gmf_user_bias,
           gmf_item_bias, mlp_user_emb, mlp_item_emb, W1, b1, W2, b2, Wf, bf):
    u2 = user_ids.astype(jnp.int32).reshape(IDX_ROWS, CHUNK)
    i2 = item_ids.astype(jnp.int32).reshape(IDX_ROWS, CHUNK)
    gu, gi, bu, bi, mu, mi = _sc_gather(
        u2, i2,
        gmf_user_emb.T.reshape(-1), gmf_item_emb.T.reshape(-1),
        gmf_user_bias, gmf_item_bias,
        mlp_user_emb.T.reshape(-1), mlp_item_emb.T.reshape(-1))
    scpair = jnp.concatenate([Wf[0:1, 0], bf]).reshape(1, 2)
    pred = _tc_mlp(
        gu, gi, bu, bi, mu, mi,
        W1[:MLP_DIM].T, W1[MLP_DIM:].T, b1.reshape(MLP_DIM, 1),
        W2.T, b2.reshape(16, 1), Wf[1:].reshape(1, 16), scpair)
    return pred.reshape(-1)
